# baseline (speedup 1.0000x reference)
import functools

import jax
import jax.numpy as jnp
from jax import lax
from jax.experimental import pallas as pl
from jax.experimental.pallas import tpu as pltpu
from jax.experimental.pallas import tpu_sc as plsc

NC = 2
NS = 16
L = 16
NW = NC * NS
B = 16384
BPW = B // NW
CHUNK = 128
NCHUNK = BPW // CHUNK
D = 32
GROUPS = BPW // L
STRIDE = L + 1
V = 1000000
NBLK = V // 128
TAIL = V - NBLK * 128
HALF = (NBLK // NW + 1 + 1) // 2


def _sc_relayout(app_t, ent_t, tail_app, tail_ent):
  mesh = plsc.VectorSubcoreMesh(
      core_axis_name="c", subcore_axis_name="s", num_cores=NC, num_subcores=NS)

  @functools.partial(
      pl.kernel,
      out_type=(jax.ShapeDtypeStruct((V * D,), jnp.float32),
                jax.ShapeDtypeStruct((V * D,), jnp.float32)),
      mesh=mesh,
      compiler_params=pltpu.CompilerParams(needs_layout_passes=False),
      scratch_types=[
          pltpu.VMEM((D, 128), jnp.float32),
          pltpu.VMEM((D, 128), jnp.float32),
          pltpu.VMEM((128 * D,), jnp.float32),
          pltpu.VMEM((128 * D,), jnp.float32),
          pltpu.VMEM((D, TAIL), jnp.float32),
          pltpu.VMEM((TAIL * D,), jnp.float32),
          pltpu.SemaphoreType.DMA,
          pltpu.SemaphoreType.DMA,
          pltpu.SemaphoreType.DMA,
      ],
  )
  def k(app_in, ent_in, ta_in, te_in, app_out, ent_out,
        in_a, in_b, out_a, out_b, tl_i, tl_o, sem_a, sem_b, osem):
    wid = lax.axis_index("s") * NC + lax.axis_index("c")
    lane = lax.iota(jnp.int32, L)

    def transpose_store(ibuf, obuf, ncols):
      for c in range(ncols):
        cs = jnp.full((L,), c, jnp.int32)
        obuf[pl.ds(c * D, L)] = plsc.load_gather(ibuf, [lane, cs])
        obuf[pl.ds(c * D + L, L)] = plsc.load_gather(ibuf, [lane + L, cs])

    def make_body(tab_in, tab_out):
      def body(t, _):
        blk_a = wid + NW * (2 * t)
        blk_b = wid + NW * (2 * t + 1)

        @pl.when(blk_a < NBLK)
        def _():
          cp_a = pltpu.async_copy(
              tab_in.at[:, pl.ds(blk_a * 128, 128)], in_a, sem_a)

          @pl.when(blk_b < NBLK)
          def _():
            pltpu.async_copy(
                tab_in.at[:, pl.ds(blk_b * 128, 128)], in_b, sem_b)

          cp_a.wait()
          transpose_store(in_a, out_a, 128)
          pltpu.async_copy(
              out_a, tab_out.at[pl.ds(blk_a * 128 * D, 128 * D)], osem).wait()

          @pl.when(blk_b < NBLK)
          def _():
            pltpu.make_async_copy(
                tab_in.at[:, pl.ds(blk_b * 128, 128)], in_b, sem_b).wait()
            transpose_store(in_b, out_b, 128)
            pltpu.async_copy(
                out_b, tab_out.at[pl.ds(blk_b * 128 * D, 128 * D)],
                osem).wait()
        return 0

      return body

    lax.fori_loop(0, HALF, make_body(app_in, app_out), 0)
    lax.fori_loop(0, HALF, make_body(ent_in, ent_out), 0)

    @pl.when(wid == 0)
    def _():
      pltpu.sync_copy(ta_in, tl_i)
      transpose_store(tl_i, tl_o, TAIL)
      pltpu.sync_copy(tl_o, app_out.at[pl.ds(NBLK * 128 * D, TAIL * D)])
      pltpu.sync_copy(te_in, tl_i)
      transpose_store(tl_i, tl_o, TAIL)
      pltpu.sync_copy(tl_o, ent_out.at[pl.ds(NBLK * 128 * D, TAIL * D)])

  return k(app_t, ent_t, tail_app, tail_ent)


def _sc_scores(ri, app_rm, ent_rm):
  mesh = plsc.VectorSubcoreMesh(
      core_axis_name="c", subcore_axis_name="s", num_cores=NC, num_subcores=NS)

  @functools.partial(
      pl.kernel,
      out_type=jax.ShapeDtypeStruct((2, NW, GROUPS, L), jnp.float32),
      mesh=mesh,
      compiler_params=pltpu.CompilerParams(
          needs_layout_passes=False, use_tc_tiling_on_sc=False),
      scratch_types=[
          pltpu.VMEM((NCHUNK, CHUNK), jnp.int32),
          pltpu.VMEM((NCHUNK, CHUNK), jnp.int32),
          pltpu.VMEM((NCHUNK, CHUNK), jnp.int32),
          pltpu.VMEM((NCHUNK, CHUNK), jnp.int32),
          pltpu.VMEM((BPW, D), jnp.float32),
          pltpu.VMEM((BPW, D), jnp.float32),
          pltpu.VMEM((BPW, D), jnp.float32),
          pltpu.VMEM((BPW, D), jnp.float32),
          pltpu.VMEM((BPW * STRIDE,), jnp.float32),
          pltpu.VMEM((BPW * STRIDE,), jnp.float32),
          pltpu.VMEM((GROUPS, L), jnp.float32),
          pltpu.VMEM((GROUPS, L), jnp.float32),
          pltpu.SemaphoreType.DMA,
      ],
  )
  def k(ri_pa, ri_pe, ri_na, ri_ne, app, ent, out_h,
        pa_i, pe_i, na_i, ne_i, pa_r, pe_r, na_r, ne_r,
        sp_flat, sn_flat, s_pos, s_neg, sem):
    wid = lax.axis_index("s") * NC + lax.axis_index("c")

    row0 = wid * NCHUNK
    pltpu.sync_copy(ri_pa.at[pl.ds(row0, NCHUNK)], pa_i)
    pltpu.sync_copy(ri_pe.at[pl.ds(row0, NCHUNK)], pe_i)
    pltpu.sync_copy(ri_na.at[pl.ds(row0, NCHUNK)], na_i)
    pltpu.sync_copy(ri_ne.at[pl.ds(row0, NCHUNK)], ne_i)

    copies = []
    for c in range(NCHUNK):
      dst = pl.ds(c * CHUNK, CHUNK)
      copies.append(pltpu.async_copy(app.at[pa_i.at[c]], pa_r.at[dst], sem))
      copies.append(pltpu.async_copy(ent.at[pe_i.at[c]], pe_r.at[dst], sem))
      copies.append(pltpu.async_copy(app.at[na_i.at[c]], na_r.at[dst], sem))
      copies.append(pltpu.async_copy(ent.at[ne_i.at[c]], ne_r.at[dst], sem))
    for cp in copies:
      cp.wait()

    lane = lax.iota(jnp.int32, L)
    lo = pl.ds(0, L)
    hi = pl.ds(L, L)

    def stage(r, _):
      sp_flat[pl.ds(r * STRIDE, L)] = (
          pa_r[r, lo] * pe_r[r, lo] + pa_r[r, hi] * pe_r[r, hi])
      sn_flat[pl.ds(r * STRIDE, L)] = (
          na_r[r, lo] * ne_r[r, lo] + na_r[r, hi] * ne_r[r, hi])
      return 0

    lax.fori_loop(0, BPW, stage, 0)

    def accum(g, _):
      base = (g * L + lane) * STRIDE
      accp = jnp.zeros((L,), jnp.float32)
      accn = jnp.zeros((L,), jnp.float32)
      for j in range(L):
        accp += plsc.load_gather(sp_flat, [base + j])
        accn += plsc.load_gather(sn_flat, [base + j])
      s_pos[g, :] = accp
      s_neg[g, :] = accn
      return 0

    lax.fori_loop(0, GROUPS, accum, 0)

    pltpu.sync_copy(s_pos, out_h.at[0, wid])
    pltpu.sync_copy(s_neg, out_h.at[1, wid])

  return k(ri[0], ri[1], ri[2], ri[3], app_rm, ent_rm)


def _tc_reduce(scores):
  def body(x_ref, o_ref):
    x = x_ref[...]
    row = lax.broadcasted_iota(jnp.int32, x.shape, 0)
    s = jnp.where(row < 128, x, -x)
    ls = jnp.minimum(s, 0.0) - jnp.log1p(jnp.exp(-jnp.abs(s)))
    o_ref[0, 0] = -jnp.sum(ls)

  out = pl.pallas_call(
      body,
      out_shape=jax.ShapeDtypeStruct((1, 1), jnp.float32),
      out_specs=pl.BlockSpec(memory_space=pltpu.SMEM),
  )(scores)
  return out[0, 0]


def kernel(pos_app, pos_entity, neg_app, neg_entity, app_emb, entity_emb):
  idx = [x.astype(jnp.int32).reshape(B // CHUNK, CHUNK)
         for x in (pos_app, pos_entity, neg_app, neg_entity)]
  app_flat, ent_flat = _sc_relayout(
      app_emb.T, entity_emb.T,
      app_emb.T[:, NBLK * 128:], entity_emb.T[:, NBLK * 128:])
  scores = _sc_scores(idx, app_flat.reshape(V, D), ent_flat.reshape(V, D))
  return _tc_reduce(scores.reshape(2 * B // 128, 128))


# speedup vs baseline: 2.0637x; 2.0637x over previous
import functools

import jax
import jax.numpy as jnp
from jax import lax
from jax.experimental import pallas as pl
from jax.experimental.pallas import tpu as pltpu
from jax.experimental.pallas import tpu_sc as plsc

NC = 2
NS = 16
L = 16
NW = NC * NS
B = 16384
BPW = B // NW
CHUNK = 128
NCHUNK = BPW // CHUNK
D = 32
GROUPS = BPW // L
STRIDE = L + 1


def _sc_scores(ri, app_rm, ent_rm):
  mesh = plsc.VectorSubcoreMesh(
      core_axis_name="c", subcore_axis_name="s", num_cores=NC, num_subcores=NS)

  @functools.partial(
      pl.kernel,
      out_type=jax.ShapeDtypeStruct((2, NW, GROUPS, L), jnp.float32),
      mesh=mesh,
      compiler_params=pltpu.CompilerParams(
          needs_layout_passes=False, use_tc_tiling_on_sc=False),
      scratch_types=[
          pltpu.VMEM((NCHUNK, CHUNK), jnp.int32),
          pltpu.VMEM((NCHUNK, CHUNK), jnp.int32),
          pltpu.VMEM((NCHUNK, CHUNK), jnp.int32),
          pltpu.VMEM((NCHUNK, CHUNK), jnp.int32),
          pltpu.VMEM((BPW, D), jnp.float32),
          pltpu.VMEM((BPW, D), jnp.float32),
          pltpu.VMEM((BPW, D), jnp.float32),
          pltpu.VMEM((BPW, D), jnp.float32),
          pltpu.VMEM((BPW * STRIDE,), jnp.float32),
          pltpu.VMEM((BPW * STRIDE,), jnp.float32),
          pltpu.VMEM((GROUPS, L), jnp.float32),
          pltpu.VMEM((GROUPS, L), jnp.float32),
          pltpu.SemaphoreType.DMA,
      ],
  )
  def k(ri_pa, ri_pe, ri_na, ri_ne, app, ent, out_h,
        pa_i, pe_i, na_i, ne_i, pa_r, pe_r, na_r, ne_r,
        sp_flat, sn_flat, s_pos, s_neg, sem):
    wid = lax.axis_index("s") * NC + lax.axis_index("c")

    row0 = wid * NCHUNK
    pltpu.sync_copy(ri_pa.at[pl.ds(row0, NCHUNK)], pa_i)
    pltpu.sync_copy(ri_pe.at[pl.ds(row0, NCHUNK)], pe_i)
    pltpu.sync_copy(ri_na.at[pl.ds(row0, NCHUNK)], na_i)
    pltpu.sync_copy(ri_ne.at[pl.ds(row0, NCHUNK)], ne_i)

    copies = []
    for c in range(NCHUNK):
      dst = pl.ds(c * CHUNK, CHUNK)
      copies.append(pltpu.async_copy(app.at[pa_i.at[c]], pa_r.at[dst], sem))
      copies.append(pltpu.async_copy(ent.at[pe_i.at[c]], pe_r.at[dst], sem))
      copies.append(pltpu.async_copy(app.at[na_i.at[c]], na_r.at[dst], sem))
      copies.append(pltpu.async_copy(ent.at[ne_i.at[c]], ne_r.at[dst], sem))
    for cp in copies:
      cp.wait()

    lane = lax.iota(jnp.int32, L)
    lo = pl.ds(0, L)
    hi = pl.ds(L, L)

    def stage(r, _):
      sp_flat[pl.ds(r * STRIDE, L)] = (
          pa_r[r, lo] * pe_r[r, lo] + pa_r[r, hi] * pe_r[r, hi])
      sn_flat[pl.ds(r * STRIDE, L)] = (
          na_r[r, lo] * ne_r[r, lo] + na_r[r, hi] * ne_r[r, hi])
      return 0

    lax.fori_loop(0, BPW, stage, 0)

    def accum(g, _):
      base = (g * L + lane) * STRIDE
      accp = jnp.zeros((L,), jnp.float32)
      accn = jnp.zeros((L,), jnp.float32)
      for j in range(L):
        accp += plsc.load_gather(sp_flat, [base + j])
        accn += plsc.load_gather(sn_flat, [base + j])
      s_pos[g, :] = accp
      s_neg[g, :] = accn
      return 0

    lax.fori_loop(0, GROUPS, accum, 0)

    pltpu.sync_copy(s_pos, out_h.at[0, wid])
    pltpu.sync_copy(s_neg, out_h.at[1, wid])

  return k(ri[0], ri[1], ri[2], ri[3], app_rm, ent_rm)


def _tc_reduce(scores):
  def body(x_ref, o_ref):
    x = x_ref[...]
    row = lax.broadcasted_iota(jnp.int32, x.shape, 0)
    s = jnp.where(row < 128, x, -x)
    ls = jnp.minimum(s, 0.0) - jnp.log1p(jnp.exp(-jnp.abs(s)))
    o_ref[0, 0] = -jnp.sum(ls)

  out = pl.pallas_call(
      body,
      out_shape=jax.ShapeDtypeStruct((1, 1), jnp.float32),
      out_specs=pl.BlockSpec(memory_space=pltpu.SMEM),
  )(scores)
  return out[0, 0]


def kernel(pos_app, pos_entity, neg_app, neg_entity, app_emb, entity_emb):
  idx = [x.astype(jnp.int32).reshape(B // CHUNK, CHUNK)
         for x in (pos_app, pos_entity, neg_app, neg_entity)]
  scores = _sc_scores(idx, app_emb, entity_emb)
  return _tc_reduce(scores.reshape(2 * B // 128, 128))
